# Initial kernel scaffold; baseline (speedup 1.0000x reference)
import jax
import jax.numpy as jnp
from jax.experimental import pallas as pl


def kernel(x, emb_tables, compress_W, compress_b, proj_W, proj_b, codebooks, sem_tables, qkv_W, qkv_b, o_W, o_b, ln1_g, ln1_b, ff_W1, ff_b1, ff_W2, ff_b2, ln2_g, ln2_b, head_W1, head_b1, head_W2, head_b2):
    raise NotImplementedError("write your pallas kernel here")


# speedup vs baseline: 3.7127x; 3.7127x over previous
import functools
import math

import jax
import jax.numpy as jnp
from jax import lax
from jax.experimental import pallas as pl
from jax.experimental.pallas import tpu as pltpu
from jax.experimental.pallas import tpu_sc as plsc

NF = 26
V = 1001
D = 128
HID = 512
NC = 4
CS = 1024
NH = 4
NL = 2
DH = D // NH


def _sc_gather_body(n_chunks, chunk, tbl_hbm, idx_hbm, out_hbm, idx_v, rows_v, sem):
    c = lax.axis_index("c")
    s = lax.axis_index("s")
    wid = s * 2 + c
    rows_per_w = n_chunks * chunk
    pltpu.sync_copy(idx_hbm.at[wid], idx_v)
    base = wid * rows_per_w

    @pl.loop(0, n_chunks)
    def _(ch):
        pltpu.async_copy(tbl_hbm.at[idx_v.at[ch]], rows_v, sem).wait()
        pltpu.sync_copy(rows_v, out_hbm.at[pl.ds(base + ch * chunk, chunk)])


def _sc_gather(flat_tables, idx3, total_rows, n_chunks, chunk):
    mesh = plsc.VectorSubcoreMesh(core_axis_name="c", subcore_axis_name="s")
    kern = functools.partial(
        pl.kernel,
        mesh=mesh,
        out_type=jax.ShapeDtypeStruct((total_rows, D), jnp.float32),
        scratch_types=[
            pltpu.VMEM((n_chunks, chunk), jnp.int32),
            pltpu.VMEM((chunk, D), jnp.float32),
            pltpu.SemaphoreType.DMA,
        ],
        compiler_params=pltpu.CompilerParams(use_tc_tiling_on_sc=False),
    )(functools.partial(_sc_gather_body, n_chunks, chunk))
    return kern(flat_tables, idx3)


def _dot(a, b, dims, precision=None):
    return lax.dot_general(a, b, (dims, ((), ())),
                           preferred_element_type=jnp.float32,
                           precision=precision)


_HI = lax.Precision.HIGHEST


def _ln(t, g, b):
    mu = jnp.mean(t, axis=1, keepdims=True)
    d = t - mu
    var = jnp.mean(d * d, axis=1, keepdims=True)
    return d * jax.lax.rsqrt(var + 1e-5) * g + b


def _tc_body(bsz, emb_ref, cW, cb, pW, pb, cbk, semt, qkvW, qkvb, oW, ob,
             ln1g, ln1b, ffW1, ffb1, ffW2, ffb2, ln2g, ln2b, hW1, hb1, hW2,
             hb2, logits_ref, loss_ref):
    step = pl.program_id(0)

    E = emb_ref[...]
    Ecat = jnp.concatenate([E[f] for f in range(NF)], axis=1)
    comp = _dot(Ecat, cW[...], (((1,), (1,))))
    comp = jnp.maximum(comp + cb[...], 0.0)

    S = comp.shape[0]
    iota_cs = lax.broadcasted_iota(jnp.int32, (S, CS), 1)
    loss_acc = jnp.float32(0.0)
    hpos = []
    for i in range(NC):
        z = _dot(comp, pW[i], (((1,), (1,)))) + pb[i, 0:1, :]
        cbi = cbk[i]
        zc = _dot(z, cbi, (((1,), (1,))))
        z2 = jnp.sum(z * z, axis=1, keepdims=True)
        c2 = jnp.sum(cbi * cbi, axis=1)[None, :]
        dist = z2 - 2.0 * zc + c2
        md = jnp.min(dist, axis=1, keepdims=True)
        loss_acc = loss_acc + jnp.sum(md)
        ind = jnp.min(jnp.where(dist == md, iota_cs, CS), axis=1,
                      keepdims=True)
        onehot = (iota_cs == ind).astype(jnp.float32)
        hpos.append(_dot(onehot, semt[i], (((1,), (0,)))))

    lane = lax.broadcasted_iota(jnp.int32, (D, NH), 0) // DH
    hsel = lax.broadcasted_iota(jnp.int32, (D, NH), 1)
    Hsum = (lane == hsel).astype(jnp.float32)
    HsumT = (lax.broadcasted_iota(jnp.int32, (NH, D), 1) // DH ==
             lax.broadcasted_iota(jnp.int32, (NH, D), 0)).astype(jnp.float32)

    scale = 1.0 / math.sqrt(DH)
    for l in range(NL):
        Wq = qkvW[l, 0:D, :]
        Wk = qkvW[l, D:2 * D, :]
        Wv = qkvW[l, 2 * D:3 * D, :]
        bq = qkvb[l, 0:1, :]
        bk = qkvb[l, 1:2, :]
        bv = qkvb[l, 2:3, :]
        q = [_dot(hpos[i], Wq, (((1,), (1,)))) + bq for i in range(NC)]
        k = [_dot(hpos[i], Wk, (((1,), (1,)))) + bk for i in range(NC)]
        v = [_dot(hpos[i], Wv, (((1,), (1,)))) + bv for i in range(NC)]
        newh = []
        for i in range(NC):
            s_ij = [_dot(q[i] * k[j], Hsum, (((1,), (0,)))) * scale
                    for j in range(NC)]
            m = jnp.maximum(jnp.maximum(s_ij[0], s_ij[1]),
                            jnp.maximum(s_ij[2], s_ij[3]))
            e_ij = [jnp.exp(sj - m) for sj in s_ij]
            den = e_ij[0] + e_ij[1] + e_ij[2] + e_ij[3]
            ao = jnp.zeros_like(hpos[i])
            for j in range(NC):
                att = e_ij[j] / den
                ao = ao + _dot(att, HsumT, (((1,), (0,)))) * v[j]
            ao = _dot(ao, oW[l], (((1,), (1,)))) + ob[l, 0:1, :]
            hh = _ln(hpos[i] + ao, ln1g[l, 0:1, :], ln1b[l, 0:1, :])
            ff = jnp.maximum(_dot(hh, ffW1[l], (((1,), (1,)))) +
                             ffb1[l, 0:1, :], 0.0)
            ff = _dot(ff, ffW2[l], (((1,), (1,)))) + ffb2[l, 0:1, :]
            newh.append(_ln(hh + ff, ln2g[l, 0:1, :], ln2b[l, 0:1, :]))
        hpos = newh

    h1 = hb1[...]
    acc = jnp.zeros((S, HID), jnp.float32)
    for i in range(NC):
        acc = acc + _dot(hpos[i], hW1[:, i * D:(i + 1) * D], (((1,), (1,))))
    h1 = jnp.maximum(acc + h1, 0.0)
    logits = jnp.sum(h1 * hW2[...], axis=1, keepdims=True) + hb2[0, 0]
    logits_ref[...] = logits

    @pl.when(step == 0)
    def _():
        loss_ref[...] = jnp.zeros_like(loss_ref)

    loss_ref[...] += (loss_acc * (1.25 / (bsz * D))).reshape(1, 1)


def kernel(x, emb_tables, compress_W, compress_b, proj_W, proj_b, codebooks,
           sem_tables, qkv_W, qkv_b, o_W, o_b, ln1_g, ln1_b, ff_W1, ff_b1,
           ff_W2, ff_b2, ln2_g, ln2_b, head_W1, head_b1, head_W2, head_b2):
    bsz = x.shape[0]
    NW = 32
    total_rows = NF * bsz
    rows_per_w = total_rows // NW
    chunk = 128
    n_chunks = rows_per_w // chunk

    idx = jnp.clip(x, 0, V - 1).astype(jnp.int32)
    flat_idx = (idx.T + (jnp.arange(NF, dtype=jnp.int32) * V)[:, None])
    idx3 = flat_idx.reshape(NW, n_chunks, chunk)
    flat_tables = emb_tables.reshape(NF * V, D)

    emb_flat = _sc_gather(flat_tables, idx3, total_rows, n_chunks, chunk)
    embeds = emb_flat.reshape(NF, bsz, D)

    S = 256
    grid = (bsz // S,)
    full = lambda *shape: pl.BlockSpec(shape, lambda i: (0,) * len(shape))

    out_shape = [
        jax.ShapeDtypeStruct((bsz, 1), jnp.float32),
        jax.ShapeDtypeStruct((1, 1), jnp.float32),
    ]
    in_specs = [
        pl.BlockSpec((NF, S, D), lambda i: (0, i, 0)),
        full(HID, NF * D),
        full(1, HID),
        full(NC, D, HID),
        full(NC, 1, D),
        full(NC, CS, D),
        full(NC, CS, D),
        full(NL, 3 * D, D),
        full(NL, 3, D),
        full(NL, D, D),
        full(NL, 1, D),
        full(NL, 1, D),
        full(NL, 1, D),
        full(NL, HID, D),
        full(NL, 1, HID),
        full(NL, D, HID),
        full(NL, 1, D),
        full(NL, 1, D),
        full(NL, 1, D),
        full(HID, NC * D),
        full(1, HID),
        full(1, HID),
        pl.BlockSpec(memory_space=pltpu.SMEM),
    ]
    out_specs = [
        pl.BlockSpec((S, 1), lambda i: (i, 0)),
        pl.BlockSpec((1, 1), lambda i: (0, 0)),
    ]

    logits, loss = pl.pallas_call(
        functools.partial(_tc_body, bsz),
        grid=grid,
        in_specs=in_specs,
        out_specs=out_specs,
        out_shape=out_shape,
    )(
        embeds,
        compress_W,
        compress_b.reshape(1, HID),
        proj_W,
        proj_b.reshape(NC, 1, D),
        codebooks,
        sem_tables,
        qkv_W,
        qkv_b.reshape(NL, 3, D),
        o_W,
        o_b.reshape(NL, 1, D),
        ln1_g.reshape(NL, 1, D),
        ln1_b.reshape(NL, 1, D),
        ff_W1,
        ff_b1.reshape(NL, 1, HID),
        ff_W2,
        ff_b2.reshape(NL, 1, D),
        ln2_g.reshape(NL, 1, D),
        ln2_b.reshape(NL, 1, D),
        head_W1,
        head_b1.reshape(1, HID),
        head_W2,
        head_b2.reshape(1, 1),
    )
    return logits[:, 0], loss[0, 0]


# speedup vs baseline: 5.0407x; 1.3577x over previous
import functools
import math

import jax
import jax.numpy as jnp
from jax import lax
from jax.experimental import pallas as pl
from jax.experimental.pallas import tpu as pltpu
from jax.experimental.pallas import tpu_sc as plsc

NF = 26
V = 1001
D = 128
HID = 512
NC = 4
CS = 1024
NH = 4
NL = 2
DH = D // NH


def _sc_gather_body(n_chunks, chunk, tbl_hbm, idx_hbm, out_hbm, idx_v, rows_v, sem):
    c = lax.axis_index("c")
    s = lax.axis_index("s")
    wid = s * 2 + c
    rows_per_w = n_chunks * chunk
    pltpu.sync_copy(idx_hbm.at[wid], idx_v)
    base = wid * rows_per_w

    @pl.loop(0, n_chunks)
    def _(ch):
        pltpu.async_copy(tbl_hbm.at[idx_v.at[ch]], rows_v, sem).wait()
        pltpu.sync_copy(rows_v, out_hbm.at[pl.ds(base + ch * chunk, chunk)])


def _sc_gather(flat_tables, idx3, total_rows, n_chunks, chunk):
    mesh = plsc.VectorSubcoreMesh(core_axis_name="c", subcore_axis_name="s")
    kern = functools.partial(
        pl.kernel,
        mesh=mesh,
        out_type=jax.ShapeDtypeStruct((total_rows, D), jnp.float32),
        scratch_types=[
            pltpu.VMEM((n_chunks, chunk), jnp.int32),
            pltpu.VMEM((chunk, D), jnp.float32),
            pltpu.SemaphoreType.DMA,
        ],
        compiler_params=pltpu.CompilerParams(use_tc_tiling_on_sc=False),
    )(functools.partial(_sc_gather_body, n_chunks, chunk))
    return kern(flat_tables, idx3)


def _dot(a, b, dims, precision=None):
    return lax.dot_general(a, b, (dims, ((), ())),
                           preferred_element_type=jnp.float32,
                           precision=precision)


_HI = lax.Precision.HIGHEST


def _ln(t, g, b):
    mu = jnp.mean(t, axis=1, keepdims=True)
    d = t - mu
    var = jnp.mean(d * d, axis=1, keepdims=True)
    return d * jax.lax.rsqrt(var + 1e-5) * g + b


def _tc_body(bsz, emb_ref, cW, cb, pW, pb, cbk, semt, qkvW, qkvb, oW, ob,
             ln1g, ln1b, ffW1, ffb1, ffW2, ffb2, ln2g, ln2b, hW1, hb1, hW2,
             hb2, logits_ref, loss_ref, c2_s):
    step = pl.program_id(0)

    @pl.when(step == 0)
    def _():
        for i in range(NC):
            c2_s[i:i + 1, :] = jnp.sum(cbk[i] * cbk[i], axis=1)[None, :]

    E = emb_ref[...]
    Ecat = jnp.concatenate([E[f] for f in range(NF)], axis=1)
    comp = _dot(Ecat, cW[...], (((1,), (1,))))
    comp = jnp.maximum(comp + cb[...], 0.0)

    S = comp.shape[0]
    iota_cs = lax.broadcasted_iota(jnp.int32, (S, CS), 1)
    loss_acc = jnp.float32(0.0)
    hpos = []
    for i in range(NC):
        z = _dot(comp, pW[i], (((1,), (1,)))) + pb[i, 0:1, :]
        cbi = cbk[i]
        zc = _dot(z, cbi, (((1,), (1,))))
        z2 = jnp.sum(z * z, axis=1, keepdims=True)
        c2 = c2_s[i:i + 1, :]
        dist = z2 - 2.0 * zc + c2
        md = jnp.min(dist, axis=1, keepdims=True)
        loss_acc = loss_acc + jnp.sum(md)
        ind = jnp.min(jnp.where(dist == md, iota_cs, CS), axis=1,
                      keepdims=True)
        onehot = (iota_cs == ind).astype(jnp.float32)
        hpos.append(_dot(onehot, semt[i], (((1,), (0,)))))

    r0 = lax.broadcasted_iota(jnp.int32, (NC * D, NC * NH), 0)
    c0 = lax.broadcasted_iota(jnp.int32, (NC * D, NC * NH), 1)
    HsumBD = ((r0 // D == c0 // NH) &
              ((r0 % D) // DH == c0 % NH)).astype(jnp.float32)
    r1 = lax.broadcasted_iota(jnp.int32, (NC * NH, NC * D), 0)
    c1 = lax.broadcasted_iota(jnp.int32, (NC * NH, NC * D), 1)
    E16 = ((c1 // D == r1 // NH) &
           ((c1 % D) // DH == r1 % NH)).astype(jnp.float32)

    scale = 1.0 / math.sqrt(DH)
    H = jnp.concatenate(hpos, axis=0)
    for l in range(NL):
        qkv = _dot(H, qkvW[l], (((1,), (1,)))) + qkvb[l]
        q = qkv[:, 0:D]
        k = qkv[:, D:2 * D]
        v = qkv[:, 2 * D:3 * D]
        Vcat = jnp.concatenate([v[j * S:(j + 1) * S] for j in range(NC)],
                               axis=1)
        aos = []
        for i in range(NC):
            qi = q[i * S:(i + 1) * S]
            P = jnp.concatenate([qi * k[j * S:(j + 1) * S]
                                 for j in range(NC)], axis=1)
            s = _dot(P, HsumBD, (((1,), (0,)))) * scale
            e = jnp.exp(s)
            den = (e[:, 0:NH] + e[:, NH:2 * NH] + e[:, 2 * NH:3 * NH] +
                   e[:, 3 * NH:4 * NH])
            dent = jnp.concatenate([den] * NC, axis=1)
            att = e / dent
            aoc = _dot(att, E16, (((1,), (0,)))) * Vcat
            aos.append(aoc[:, 0:D] + aoc[:, D:2 * D] +
                       aoc[:, 2 * D:3 * D] + aoc[:, 3 * D:4 * D])
        AO = jnp.concatenate(aos, axis=0)
        AO = _dot(AO, oW[l], (((1,), (1,)))) + ob[l, 0:1, :]
        H = _ln(H + AO, ln1g[l, 0:1, :], ln1b[l, 0:1, :])
        ff = jnp.maximum(_dot(H, ffW1[l], (((1,), (1,)))) +
                         ffb1[l, 0:1, :], 0.0)
        ff = _dot(ff, ffW2[l], (((1,), (1,)))) + ffb2[l, 0:1, :]
        H = _ln(H + ff, ln2g[l, 0:1, :], ln2b[l, 0:1, :])

    h1 = hb1[...]
    acc = jnp.zeros((S, HID), jnp.float32)
    for i in range(NC):
        acc = acc + _dot(H[i * S:(i + 1) * S], hW1[:, i * D:(i + 1) * D],
                         (((1,), (1,))))
    h1 = jnp.maximum(acc + h1, 0.0)
    logits = jnp.sum(h1 * hW2[...], axis=1, keepdims=True) + hb2[0, 0]
    logits_ref[...] = logits

    @pl.when(step == 0)
    def _():
        loss_ref[...] = jnp.zeros_like(loss_ref)

    loss_ref[...] += (loss_acc * (1.25 / (bsz * D))).reshape(1, 1)


def kernel(x, emb_tables, compress_W, compress_b, proj_W, proj_b, codebooks,
           sem_tables, qkv_W, qkv_b, o_W, o_b, ln1_g, ln1_b, ff_W1, ff_b1,
           ff_W2, ff_b2, ln2_g, ln2_b, head_W1, head_b1, head_W2, head_b2):
    bsz = x.shape[0]
    NW = 32
    total_rows = NF * bsz
    rows_per_w = total_rows // NW
    chunk = 128
    n_chunks = rows_per_w // chunk

    idx = jnp.clip(x, 0, V - 1).astype(jnp.int32)
    flat_idx = (idx.T + (jnp.arange(NF, dtype=jnp.int32) * V)[:, None])
    idx3 = flat_idx.reshape(NW, n_chunks, chunk)
    flat_tables = emb_tables.reshape(NF * V, D)

    emb_flat = _sc_gather(flat_tables, idx3, total_rows, n_chunks, chunk)
    embeds = emb_flat.reshape(NF, bsz, D)

    S = 512
    grid = (bsz // S,)
    full = lambda *shape: pl.BlockSpec(shape, lambda i: (0,) * len(shape))

    out_shape = [
        jax.ShapeDtypeStruct((bsz, 1), jnp.float32),
        jax.ShapeDtypeStruct((1, 1), jnp.float32),
    ]
    in_specs = [
        pl.BlockSpec((NF, S, D), lambda i: (0, i, 0)),
        full(HID, NF * D),
        full(1, HID),
        full(NC, D, HID),
        full(NC, 1, D),
        full(NC, CS, D),
        full(NC, CS, D),
        full(NL, 3 * D, D),
        full(NL, 1, 3 * D),
        full(NL, D, D),
        full(NL, 1, D),
        full(NL, 1, D),
        full(NL, 1, D),
        full(NL, HID, D),
        full(NL, 1, HID),
        full(NL, D, HID),
        full(NL, 1, D),
        full(NL, 1, D),
        full(NL, 1, D),
        full(HID, NC * D),
        full(1, HID),
        full(1, HID),
        pl.BlockSpec(memory_space=pltpu.SMEM),
    ]
    out_specs = [
        pl.BlockSpec((S, 1), lambda i: (i, 0)),
        pl.BlockSpec((1, 1), lambda i: (0, 0)),
    ]

    logits, loss = pl.pallas_call(
        functools.partial(_tc_body, bsz),
        grid=grid,
        in_specs=in_specs,
        out_specs=out_specs,
        out_shape=out_shape,
        scratch_shapes=[pltpu.VMEM((NC, CS), jnp.float32)],
    )(
        embeds,
        compress_W,
        compress_b.reshape(1, HID),
        proj_W,
        proj_b.reshape(NC, 1, D),
        codebooks,
        sem_tables,
        qkv_W,
        qkv_b.reshape(NL, 1, 3 * D),
        o_W,
        o_b.reshape(NL, 1, D),
        ln1_g.reshape(NL, 1, D),
        ln1_b.reshape(NL, 1, D),
        ff_W1,
        ff_b1.reshape(NL, 1, HID),
        ff_W2,
        ff_b2.reshape(NL, 1, D),
        ln2_g.reshape(NL, 1, D),
        ln2_b.reshape(NL, 1, D),
        head_W1,
        head_b1.reshape(1, HID),
        head_W2,
        head_b2.reshape(1, 1),
    )
    return logits[:, 0], loss[0, 0]


# speedup vs baseline: 5.0433x; 1.0005x over previous
import functools
import math

import jax
import jax.numpy as jnp
from jax import lax
from jax.experimental import pallas as pl
from jax.experimental.pallas import tpu as pltpu
from jax.experimental.pallas import tpu_sc as plsc

NF = 26
V = 1001
D = 128
HID = 512
NC = 4
CS = 1024
NH = 4
NL = 2
DH = D // NH


def _sc_gather_body(n_chunks, chunk, tbl_hbm, idx_hbm, out_hbm, idx_v, rows_v, sem):
    c = lax.axis_index("c")
    s = lax.axis_index("s")
    wid = s * 2 + c
    rows_per_w = n_chunks * chunk
    pltpu.sync_copy(idx_hbm.at[wid], idx_v)
    base = wid * rows_per_w

    @pl.loop(0, n_chunks)
    def _(ch):
        pltpu.async_copy(tbl_hbm.at[idx_v.at[ch]], rows_v, sem).wait()
        pltpu.sync_copy(rows_v, out_hbm.at[pl.ds(base + ch * chunk, chunk)])


def _sc_gather(flat_tables, idx3, total_rows, n_chunks, chunk):
    mesh = plsc.VectorSubcoreMesh(core_axis_name="c", subcore_axis_name="s")
    kern = functools.partial(
        pl.kernel,
        mesh=mesh,
        out_type=jax.ShapeDtypeStruct((total_rows, D), jnp.float32),
        scratch_types=[
            pltpu.VMEM((n_chunks, chunk), jnp.int32),
            pltpu.VMEM((chunk, D), jnp.float32),
            pltpu.SemaphoreType.DMA,
        ],
        compiler_params=pltpu.CompilerParams(use_tc_tiling_on_sc=False),
    )(functools.partial(_sc_gather_body, n_chunks, chunk))
    return kern(flat_tables, idx3)


def _dot(a, b, dims, precision=None):
    return lax.dot_general(a, b, (dims, ((), ())),
                           preferred_element_type=jnp.float32,
                           precision=precision)


_HI = lax.Precision.HIGHEST


def _ln(t, g, b):
    mu = jnp.mean(t, axis=1, keepdims=True)
    d = t - mu
    var = jnp.mean(d * d, axis=1, keepdims=True)
    return d * jax.lax.rsqrt(var + 1e-5) * g + b


def _tc_body(bsz, emb_ref, cW, cb, pW, pb, cbk, semt, qkvW, qkvb, oW, ob,
             ln1g, ln1b, ffW1, ffb1, ffW2, ffb2, ln2g, ln2b, hW1, hb1, hW2,
             hb2, logits_ref, loss_ref, c2_s):
    step = pl.program_id(0)

    @pl.when(step == 0)
    def _():
        for i in range(NC):
            c2_s[i:i + 1, :] = jnp.sum(cbk[i] * cbk[i], axis=1)[None, :]

    E = emb_ref[...]
    Ecat = jnp.concatenate([E[f] for f in range(NF)], axis=1)
    comp = _dot(Ecat, cW[...], (((1,), (1,))))
    comp = jnp.maximum(comp + cb[...], 0.0)

    S = comp.shape[0]
    iota_cs = lax.broadcasted_iota(jnp.int32, (S, CS), 1)
    loss_acc = jnp.float32(0.0)
    hpos = []
    for i in range(NC):
        z = _dot(comp, pW[i], (((1,), (1,)))) + pb[i, 0:1, :]
        cbi = cbk[i]
        zc = _dot(z, cbi, (((1,), (1,))))
        z2 = jnp.sum(z * z, axis=1, keepdims=True)
        c2 = c2_s[i:i + 1, :]
        dist = z2 - 2.0 * zc + c2
        md = jnp.min(dist, axis=1, keepdims=True)
        loss_acc = loss_acc + jnp.sum(md)
        ind = jnp.min(jnp.where(dist == md, iota_cs, CS), axis=1,
                      keepdims=True)
        onehot = (iota_cs == ind).astype(jnp.float32)
        hpos.append(_dot(onehot, semt[i], (((1,), (0,)))))

    r0 = lax.broadcasted_iota(jnp.int32, (NC * D, NC * NH), 0)
    c0 = lax.broadcasted_iota(jnp.int32, (NC * D, NC * NH), 1)
    HsumBD = ((r0 // D == c0 // NH) &
              ((r0 % D) // DH == c0 % NH)).astype(jnp.float32)
    r1 = lax.broadcasted_iota(jnp.int32, (NC * NH, NC * D), 0)
    c1 = lax.broadcasted_iota(jnp.int32, (NC * NH, NC * D), 1)
    E16 = ((c1 // D == r1 // NH) &
           ((c1 % D) // DH == r1 % NH)).astype(jnp.float32)

    scale = 1.0 / math.sqrt(DH)
    H = jnp.concatenate(hpos, axis=0)
    for l in range(NL):
        qkv = _dot(H, qkvW[l], (((1,), (1,)))) + qkvb[l]
        q = qkv[:, 0:D]
        k = qkv[:, D:2 * D]
        v = qkv[:, 2 * D:3 * D]
        Vcat = jnp.concatenate([v[j * S:(j + 1) * S] for j in range(NC)],
                               axis=1)
        aos = []
        for i in range(NC):
            qi = q[i * S:(i + 1) * S]
            P = jnp.concatenate([qi * k[j * S:(j + 1) * S]
                                 for j in range(NC)], axis=1)
            s = _dot(P, HsumBD, (((1,), (0,)))) * scale
            e = jnp.exp(s)
            den = (e[:, 0:NH] + e[:, NH:2 * NH] + e[:, 2 * NH:3 * NH] +
                   e[:, 3 * NH:4 * NH])
            dent = jnp.concatenate([den] * NC, axis=1)
            att = e / dent
            aoc = _dot(att, E16, (((1,), (0,)))) * Vcat
            aos.append(aoc[:, 0:D] + aoc[:, D:2 * D] +
                       aoc[:, 2 * D:3 * D] + aoc[:, 3 * D:4 * D])
        AO = jnp.concatenate(aos, axis=0)
        AO = _dot(AO, oW[l], (((1,), (1,)))) + ob[l, 0:1, :]
        H = _ln(H + AO, ln1g[l, 0:1, :], ln1b[l, 0:1, :])
        ff = jnp.maximum(_dot(H, ffW1[l], (((1,), (1,)))) +
                         ffb1[l, 0:1, :], 0.0)
        ff = _dot(ff, ffW2[l], (((1,), (1,)))) + ffb2[l, 0:1, :]
        H = _ln(H + ff, ln2g[l, 0:1, :], ln2b[l, 0:1, :])

    h1 = hb1[...]
    acc = jnp.zeros((S, HID), jnp.float32)
    for i in range(NC):
        acc = acc + _dot(H[i * S:(i + 1) * S], hW1[:, i * D:(i + 1) * D],
                         (((1,), (1,))))
    h1 = jnp.maximum(acc + h1, 0.0)
    logits = jnp.sum(h1 * hW2[...], axis=1, keepdims=True) + hb2[0, 0]
    logits_ref[...] = logits

    @pl.when(step == 0)
    def _():
        loss_ref[...] = jnp.zeros_like(loss_ref)

    loss_ref[...] += (loss_acc * (1.25 / (bsz * D))).reshape(1, 1)


def kernel(x, emb_tables, compress_W, compress_b, proj_W, proj_b, codebooks,
           sem_tables, qkv_W, qkv_b, o_W, o_b, ln1_g, ln1_b, ff_W1, ff_b1,
           ff_W2, ff_b2, ln2_g, ln2_b, head_W1, head_b1, head_W2, head_b2):
    bsz = x.shape[0]
    NW = 32
    chunk = 128
    halves = 2
    hb = bsz // halves
    total_rows = NF * hb
    n_chunks = total_rows // NW // chunk

    idx = jnp.clip(x, 0, V - 1).astype(jnp.int32)
    flat_tables = emb_tables.reshape(NF * V, D)
    foff = (jnp.arange(NF, dtype=jnp.int32) * V)[:, None]
    embeds_h = []
    for h in range(halves):
        flat_idx = idx[h * hb:(h + 1) * hb].T + foff
        idx3 = flat_idx.reshape(NW, n_chunks, chunk)
        emb_flat = _sc_gather(flat_tables, idx3, total_rows, n_chunks, chunk)
        embeds_h.append(emb_flat.reshape(NF, hb, D))

    S = 512
    grid = (hb // S,)
    full = lambda *shape: pl.BlockSpec(shape, lambda i: (0,) * len(shape))

    out_shape = [
        jax.ShapeDtypeStruct((hb, 1), jnp.float32),
        jax.ShapeDtypeStruct((1, 1), jnp.float32),
    ]
    in_specs = [
        pl.BlockSpec((NF, S, D), lambda i: (0, i, 0)),
        full(HID, NF * D),
        full(1, HID),
        full(NC, D, HID),
        full(NC, 1, D),
        full(NC, CS, D),
        full(NC, CS, D),
        full(NL, 3 * D, D),
        full(NL, 1, 3 * D),
        full(NL, D, D),
        full(NL, 1, D),
        full(NL, 1, D),
        full(NL, 1, D),
        full(NL, HID, D),
        full(NL, 1, HID),
        full(NL, D, HID),
        full(NL, 1, D),
        full(NL, 1, D),
        full(NL, 1, D),
        full(HID, NC * D),
        full(1, HID),
        full(1, HID),
        pl.BlockSpec(memory_space=pltpu.SMEM),
    ]
    out_specs = [
        pl.BlockSpec((S, 1), lambda i: (i, 0)),
        pl.BlockSpec((1, 1), lambda i: (0, 0)),
    ]

    tc = pl.pallas_call(
        functools.partial(_tc_body, bsz),
        grid=grid,
        in_specs=in_specs,
        out_specs=out_specs,
        out_shape=out_shape,
        scratch_shapes=[pltpu.VMEM((NC, CS), jnp.float32)],
    )
    outs = [tc(
        embeds,
        compress_W,
        compress_b.reshape(1, HID),
        proj_W,
        proj_b.reshape(NC, 1, D),
        codebooks,
        sem_tables,
        qkv_W,
        qkv_b.reshape(NL, 1, 3 * D),
        o_W,
        o_b.reshape(NL, 1, D),
        ln1_g.reshape(NL, 1, D),
        ln1_b.reshape(NL, 1, D),
        ff_W1,
        ff_b1.reshape(NL, 1, HID),
        ff_W2,
        ff_b2.reshape(NL, 1, D),
        ln2_g.reshape(NL, 1, D),
        ln2_b.reshape(NL, 1, D),
        head_W1,
        head_b1.reshape(1, HID),
        head_W2,
        head_b2.reshape(1, 1),
    ) for embeds in embeds_h]
    logits = jnp.concatenate([o[0][:, 0] for o in outs])
    loss = outs[0][1][0, 0] + outs[1][1][0, 0]
    return logits, loss


# speedup vs baseline: 5.2231x; 1.0357x over previous
import functools
import math

import jax
import jax.numpy as jnp
from jax import lax
from jax.experimental import pallas as pl
from jax.experimental.pallas import tpu as pltpu
from jax.experimental.pallas import tpu_sc as plsc

NF = 26
V = 1001
D = 128
HID = 512
NC = 4
CS = 1024
NH = 4
NL = 2
DH = D // NH


def _sc_gather_body(n_chunks, chunk, tbl_hbm, idx_hbm, out_hbm, idx_v,
                    rows0, rows1, semg0, semg1, sems0, sems1):
    c = lax.axis_index("c")
    s = lax.axis_index("s")
    wid = s * 2 + c
    rows_per_w = n_chunks * chunk
    pltpu.sync_copy(idx_hbm.at[wid], idx_v)
    base = wid * rows_per_w

    bufs = [rows0, rows1]
    semg = [semg0, semg1]
    sems = [sems0, sems1]
    gath = {}
    scat = {}
    gath[0] = pltpu.async_copy(tbl_hbm.at[idx_v.at[0]], bufs[0], semg[0])
    for ch in range(n_chunks):
        nxt = ch + 1
        if nxt < n_chunks:
            if nxt >= 2:
                scat[nxt - 2].wait()
            gath[nxt] = pltpu.async_copy(tbl_hbm.at[idx_v.at[nxt]],
                                         bufs[nxt % 2], semg[nxt % 2])
        gath[ch].wait()
        scat[ch] = pltpu.async_copy(
            bufs[ch % 2], out_hbm.at[pl.ds(base + ch * chunk, chunk)],
            sems[ch % 2])
    scat[n_chunks - 1].wait()
    if n_chunks >= 2:
        scat[n_chunks - 2].wait()


def _sc_gather(flat_tables, idx3, total_rows, n_chunks, chunk):
    mesh = plsc.VectorSubcoreMesh(core_axis_name="c", subcore_axis_name="s")
    kern = functools.partial(
        pl.kernel,
        mesh=mesh,
        out_type=jax.ShapeDtypeStruct((total_rows, D), jnp.float32),
        scratch_types=[
            pltpu.VMEM((n_chunks, chunk), jnp.int32),
            pltpu.VMEM((chunk, D), jnp.float32),
            pltpu.VMEM((chunk, D), jnp.float32),
            pltpu.SemaphoreType.DMA,
            pltpu.SemaphoreType.DMA,
            pltpu.SemaphoreType.DMA,
            pltpu.SemaphoreType.DMA,
        ],
        compiler_params=pltpu.CompilerParams(use_tc_tiling_on_sc=False),
    )(functools.partial(_sc_gather_body, n_chunks, chunk))
    return kern(flat_tables, idx3)


def _dot(a, b, dims, precision=None):
    return lax.dot_general(a, b, (dims, ((), ())),
                           preferred_element_type=jnp.float32,
                           precision=precision)


_HI = lax.Precision.HIGHEST


def _ln(t, g, b):
    mu = jnp.mean(t, axis=1, keepdims=True)
    d = t - mu
    var = jnp.mean(d * d, axis=1, keepdims=True)
    return d * jax.lax.rsqrt(var + 1e-5) * g + b


def _tc_body(bsz, emb_ref, cW, cb, pW, pb, cbk, semt, qkvW, qkvb, oW, ob,
             ln1g, ln1b, ffW1, ffb1, ffW2, ffb2, ln2g, ln2b, hW1, hb1, hW2,
             hb2, logits_ref, loss_ref, c2_s):
    step = pl.program_id(0)

    @pl.when(step == 0)
    def _():
        for i in range(NC):
            c2_s[i:i + 1, :] = jnp.sum(cbk[i] * cbk[i], axis=1)[None, :]

    E = emb_ref[...]
    Ecat = jnp.concatenate([E[f] for f in range(NF)], axis=1)
    comp = _dot(Ecat, cW[...], (((1,), (1,))))
    comp = jnp.maximum(comp + cb[...], 0.0)

    S = comp.shape[0]
    iota_cs = lax.broadcasted_iota(jnp.int32, (S, CS), 1)
    loss_acc = jnp.float32(0.0)
    hpos = []
    for i in range(NC):
        z = _dot(comp, pW[i], (((1,), (1,)))) + pb[i, 0:1, :]
        cbi = cbk[i]
        zc = _dot(z, cbi, (((1,), (1,))))
        z2 = jnp.sum(z * z, axis=1, keepdims=True)
        c2 = c2_s[i:i + 1, :]
        dist = z2 - 2.0 * zc + c2
        md = jnp.min(dist, axis=1, keepdims=True)
        loss_acc = loss_acc + jnp.sum(md)
        ind = jnp.min(jnp.where(dist == md, iota_cs, CS), axis=1,
                      keepdims=True)
        onehot = (iota_cs == ind).astype(jnp.float32)
        hpos.append(_dot(onehot, semt[i], (((1,), (0,)))))

    r0 = lax.broadcasted_iota(jnp.int32, (NC * D, NC * NH), 0)
    c0 = lax.broadcasted_iota(jnp.int32, (NC * D, NC * NH), 1)
    HsumBD = ((r0 // D == c0 // NH) &
              ((r0 % D) // DH == c0 % NH)).astype(jnp.float32)
    r1 = lax.broadcasted_iota(jnp.int32, (NC * NH, NC * D), 0)
    c1 = lax.broadcasted_iota(jnp.int32, (NC * NH, NC * D), 1)
    E16 = ((c1 // D == r1 // NH) &
           ((c1 % D) // DH == r1 % NH)).astype(jnp.float32)

    scale = 1.0 / math.sqrt(DH)
    H = jnp.concatenate(hpos, axis=0)
    for l in range(NL):
        qkv = _dot(H, qkvW[l], (((1,), (1,)))) + qkvb[l]
        q = qkv[:, 0:D]
        k = qkv[:, D:2 * D]
        v = qkv[:, 2 * D:3 * D]
        Vcat = jnp.concatenate([v[j * S:(j + 1) * S] for j in range(NC)],
                               axis=1)
        aos = []
        for i in range(NC):
            qi = q[i * S:(i + 1) * S]
            P = jnp.concatenate([qi * k[j * S:(j + 1) * S]
                                 for j in range(NC)], axis=1)
            s = _dot(P, HsumBD, (((1,), (0,)))) * scale
            e = jnp.exp(s)
            den = (e[:, 0:NH] + e[:, NH:2 * NH] + e[:, 2 * NH:3 * NH] +
                   e[:, 3 * NH:4 * NH])
            dent = jnp.concatenate([den] * NC, axis=1)
            att = e / dent
            aoc = _dot(att, E16, (((1,), (0,)))) * Vcat
            aos.append(aoc[:, 0:D] + aoc[:, D:2 * D] +
                       aoc[:, 2 * D:3 * D] + aoc[:, 3 * D:4 * D])
        AO = jnp.concatenate(aos, axis=0)
        AO = _dot(AO, oW[l], (((1,), (1,)))) + ob[l, 0:1, :]
        H = _ln(H + AO, ln1g[l, 0:1, :], ln1b[l, 0:1, :])
        ff = jnp.maximum(_dot(H, ffW1[l], (((1,), (1,)))) +
                         ffb1[l, 0:1, :], 0.0)
        ff = _dot(ff, ffW2[l], (((1,), (1,)))) + ffb2[l, 0:1, :]
        H = _ln(H + ff, ln2g[l, 0:1, :], ln2b[l, 0:1, :])

    h1 = hb1[...]
    acc = jnp.zeros((S, HID), jnp.float32)
    for i in range(NC):
        acc = acc + _dot(H[i * S:(i + 1) * S], hW1[:, i * D:(i + 1) * D],
                         (((1,), (1,))))
    h1 = jnp.maximum(acc + h1, 0.0)
    logits = jnp.sum(h1 * hW2[...], axis=1, keepdims=True) + hb2[0, 0]
    logits_ref[...] = logits

    @pl.when(step == 0)
    def _():
        loss_ref[...] = jnp.zeros_like(loss_ref)

    loss_ref[...] += (loss_acc * (1.25 / (bsz * D))).reshape(1, 1)


def kernel(x, emb_tables, compress_W, compress_b, proj_W, proj_b, codebooks,
           sem_tables, qkv_W, qkv_b, o_W, o_b, ln1_g, ln1_b, ff_W1, ff_b1,
           ff_W2, ff_b2, ln2_g, ln2_b, head_W1, head_b1, head_W2, head_b2):
    bsz = x.shape[0]
    NW = 32
    chunk = 128
    halves = 2
    hb = bsz // halves
    total_rows = NF * hb
    n_chunks = total_rows // NW // chunk

    idx = jnp.clip(x, 0, V - 1).astype(jnp.int32)
    flat_tables = emb_tables.reshape(NF * V, D)
    foff = (jnp.arange(NF, dtype=jnp.int32) * V)[:, None]
    embeds_h = []
    for h in range(halves):
        flat_idx = idx[h * hb:(h + 1) * hb].T + foff
        idx3 = flat_idx.reshape(NW, n_chunks, chunk)
        emb_flat = _sc_gather(flat_tables, idx3, total_rows, n_chunks, chunk)
        embeds_h.append(emb_flat.reshape(NF, hb, D))

    S = 512
    grid = (hb // S,)
    full = lambda *shape: pl.BlockSpec(shape, lambda i: (0,) * len(shape))

    out_shape = [
        jax.ShapeDtypeStruct((hb, 1), jnp.float32),
        jax.ShapeDtypeStruct((1, 1), jnp.float32),
    ]
    in_specs = [
        pl.BlockSpec((NF, S, D), lambda i: (0, i, 0)),
        full(HID, NF * D),
        full(1, HID),
        full(NC, D, HID),
        full(NC, 1, D),
        full(NC, CS, D),
        full(NC, CS, D),
        full(NL, 3 * D, D),
        full(NL, 1, 3 * D),
        full(NL, D, D),
        full(NL, 1, D),
        full(NL, 1, D),
        full(NL, 1, D),
        full(NL, HID, D),
        full(NL, 1, HID),
        full(NL, D, HID),
        full(NL, 1, D),
        full(NL, 1, D),
        full(NL, 1, D),
        full(HID, NC * D),
        full(1, HID),
        full(1, HID),
        pl.BlockSpec(memory_space=pltpu.SMEM),
    ]
    out_specs = [
        pl.BlockSpec((S, 1), lambda i: (i, 0)),
        pl.BlockSpec((1, 1), lambda i: (0, 0)),
    ]

    tc = pl.pallas_call(
        functools.partial(_tc_body, bsz),
        grid=grid,
        in_specs=in_specs,
        out_specs=out_specs,
        out_shape=out_shape,
        scratch_shapes=[pltpu.VMEM((NC, CS), jnp.float32)],
    )
    outs = [tc(
        embeds,
        compress_W,
        compress_b.reshape(1, HID),
        proj_W,
        proj_b.reshape(NC, 1, D),
        codebooks,
        sem_tables,
        qkv_W,
        qkv_b.reshape(NL, 1, 3 * D),
        o_W,
        o_b.reshape(NL, 1, D),
        ln1_g.reshape(NL, 1, D),
        ln1_b.reshape(NL, 1, D),
        ff_W1,
        ff_b1.reshape(NL, 1, HID),
        ff_W2,
        ff_b2.reshape(NL, 1, D),
        ln2_g.reshape(NL, 1, D),
        ln2_b.reshape(NL, 1, D),
        head_W1,
        head_b1.reshape(1, HID),
        head_W2,
        head_b2.reshape(1, 1),
    ) for embeds in embeds_h]
    logits = jnp.concatenate([o[0][:, 0] for o in outs])
    loss = outs[0][1][0, 0] + outs[1][1][0, 0]
    return logits, loss
